# baseline (speedup 1.0000x reference)
import jax
import jax.numpy as jnp
from jax.experimental import pallas as pl
from jax.experimental.pallas import tpu as pltpu

_SUB = 4


def _dot(a, b, dims):
    return jax.lax.dot_general(a, b, (dims, ((), ())),
                               preferred_element_type=jnp.float32)


def _gnn_kernel(x_ref, mask_ref, w1_ref, b1_ref, wl1_ref, bl1_ref, wr1_ref,
                wl2_ref, bl2_ref, wr2_ref, out_ref):
    sub, n, hdim = x_ref.shape
    xb = x_ref[...].reshape(sub * n, hdim)
    h_all = _dot(xb, w1_ref[...], (((1,), (1,)))) + b1_ref[...]

    inv_cnt = 1.0 / (n - 1)

    for i in range(sub):
        h = h_all[i * n:(i + 1) * n]
        nrm2 = jnp.sum(h * h, axis=1, keepdims=True)
        rn = 1.0 / jnp.maximum(jnp.sqrt(nrm2), 1e-8)
        u = h * rn
        c = nrm2 * (rn * rn)

        s1 = _dot(u, _dot(u, h, (((0,), (0,)))), (((1,), (0,))))
        agg1 = (s1 - c * h) * inv_cnt
        o1 = jnp.maximum(
            _dot(agg1, wl1_ref[...], (((1,), (1,))))
            + _dot(h, wr1_ref[...], (((1,), (1,))))
            + bl1_ref[...], 0.0)

        s2 = _dot(u, _dot(u, o1, (((0,), (0,)))), (((1,), (0,))))
        agg2 = (s2 - c * o1) * inv_cnt
        z = (_dot(wl2_ref[...], agg2, (((1,), (1,))))
             + _dot(wr2_ref[...], o1, (((1,), (1,))))
             + bl2_ref[...])
        out_ref[i] = jax.nn.sigmoid(z) * mask_ref[i]


@jax.jit
def kernel(x, mask_cls, W1, b1, Wl1, bl1, Wr1, Wl2, bl2, Wr2):
    B, N, H = x.shape
    full = lambda s: pl.BlockSpec(s, lambda i: (0,) * len(s))
    out = pl.pallas_call(
        _gnn_kernel,
        grid=(B // _SUB,),
        in_specs=[
            pl.BlockSpec((_SUB, N, H), lambda i: (i, 0, 0)),
            pl.BlockSpec((_SUB, 1, N), lambda i: (i, 0, 0)),
            full(W1.shape),
            full((1, 128)),
            full(Wl1.shape),
            full((1, 64)),
            full(Wr1.shape),
            full(Wl2.shape),
            full((1, 1)),
            full(Wr2.shape),
        ],
        out_specs=pl.BlockSpec((_SUB, 1, N), lambda i: (i, 0, 0)),
        out_shape=jax.ShapeDtypeStruct((B, 1, N), jnp.float32),
        compiler_params=pltpu.CompilerParams(
            dimension_semantics=("parallel",)),
    )(x, mask_cls.reshape(B, 1, N), W1, b1.reshape(1, 128), Wl1,
      bl1.reshape(1, 64), Wr1, Wl2, bl2.reshape(1, 1), Wr2)
    return out.reshape(B, N)


# speedup vs baseline: 1.3335x; 1.3335x over previous
import jax
import jax.numpy as jnp
from jax.experimental import pallas as pl


def _dot(a, b, dims):
    return jax.lax.dot_general(a, b, (dims, ((), ())),
                               preferred_element_type=jnp.float32)


def _gnn_kernel(x_ref, mask_ref, w1_ref, b1_ref, wl1_ref, bl1_ref, wr1_ref,
                wl2_ref, bl2_ref, wr2_ref, out_ref):
    bsz, n, hdim = x_ref.shape
    xb = x_ref[...].reshape(bsz * n, hdim)
    h_all = (_dot(xb, w1_ref[...], (((1,), (1,))))
             + b1_ref[...].reshape(1, b1_ref.shape[0]))

    bl1 = bl1_ref[...].reshape(1, bl1_ref.shape[0])
    bl2 = bl2_ref[...].reshape(1, 1)
    inv_cnt = 1.0 / (n - 1)

    for i in range(bsz):
        h = h_all[i * n:(i + 1) * n]
        nrm2 = jnp.sum(h * h, axis=1, keepdims=True)
        rn = 1.0 / jnp.maximum(jnp.sqrt(nrm2), 1e-8)
        u = h * rn
        c = nrm2 * (rn * rn)

        s1 = _dot(u, _dot(u, h, (((0,), (0,)))), (((1,), (0,))))
        agg1 = (s1 - c * h) * inv_cnt
        o1 = jnp.maximum(
            _dot(agg1, wl1_ref[...], (((1,), (1,))))
            + _dot(h, wr1_ref[...], (((1,), (1,))))
            + bl1, 0.0)

        s2 = _dot(u, _dot(u, o1, (((0,), (0,)))), (((1,), (0,))))
        agg2 = (s2 - c * o1) * inv_cnt
        z = (_dot(wl2_ref[...], agg2, (((1,), (1,))))
             + _dot(wr2_ref[...], o1, (((1,), (1,))))
             + bl2)
        out_ref[i:i + 1, :] = jax.nn.sigmoid(z) * mask_ref[i:i + 1, :]


@jax.jit
def kernel(x, mask_cls, W1, b1, Wl1, bl1, Wr1, Wl2, bl2, Wr2):
    B, N, H = x.shape
    return pl.pallas_call(
        _gnn_kernel,
        out_shape=jax.ShapeDtypeStruct((B, N), jnp.float32),
    )(x, mask_cls, W1, b1, Wl1, bl1, Wr1, Wl2, bl2, Wr2)
